# Initial kernel scaffold; baseline (speedup 1.0000x reference)
import jax
import jax.numpy as jnp
from jax.experimental import pallas as pl


def kernel(coordinates, lines0, lines1, lines2, W1, b1, W2, b2):
    raise NotImplementedError("write your pallas kernel here")


# speedup vs baseline: 117.3264x; 117.3264x over previous
import functools

import jax
import jax.numpy as jnp
from jax import lax
from jax.experimental import pallas as pl
from jax.experimental.pallas import tpu as pltpu
from jax.experimental.pallas import tpu_sc as plsc

NC = 2
NS = 16
NW = NC * NS
L = 16

_RMAGIC = 12582912.0


def _sc_occupancy(coords_t, nl, scale, shift):
    total = coords_t.shape[0] // 3
    chunk = total // NW
    steps = chunk // L
    zsteps = 3 * nl // L

    mesh = plsc.VectorSubcoreMesh(core_axis_name="c", subcore_axis_name="s")

    @functools.partial(
        pl.kernel,
        out_type=jax.ShapeDtypeStruct((3 * NW * nl,), jnp.float32),
        mesh=mesh,
        compiler_params=pltpu.CompilerParams(needs_layout_passes=False),
        scratch_types=[
            pltpu.VMEM((3 * chunk,), jnp.float32),
            pltpu.VMEM((3 * nl,), jnp.float32),
        ],
    )
    def occ_kernel(coords_hbm, pocc_hbm, cbuf, occ):
        wid = lax.axis_index("s") * NC + lax.axis_index("c")
        base = wid * chunk
        for a in range(3):
            pltpu.sync_copy(coords_hbm.at[pl.ds(a * total + base, chunk)],
                            cbuf.at[pl.ds(a * chunk, chunk)])

        zero16 = jnp.zeros((L,), jnp.float32)

        def zstep(i, carry):
            occ[pl.ds(i * L, L)] = zero16
            return carry

        lax.fori_loop(0, zsteps, zstep, 0)

        ones16 = jnp.ones((L,), jnp.float32)
        hi = float(nl - 1)
        for a in range(3):
            off = a * nl

            def step(i, carry, _a=a, _off=off):
                x = cbuf[pl.ds(_a * chunk + i * L, L)]
                v = (x + shift) * scale
                vr = (v + _RMAGIC) - _RMAGIC
                mask = jnp.logical_and(vr >= 0.0, vr <= hi)
                vsafe = jnp.where(mask, vr, 0.0)
                idx = vsafe.astype(jnp.int32) + _off
                plsc.store_scatter(occ, [idx], ones16, mask=mask)
                return carry

            lax.fori_loop(0, steps, step, 0)

        for a in range(3):
            pltpu.sync_copy(occ.at[pl.ds(a * nl, nl)],
                            pocc_hbm.at[pl.ds((a * NW + wid) * nl, nl)])

    return occ_kernel(coords_t).reshape(3, NW, nl)


def _tc_mlp(occ_cols, lines0, lines1, lines2, w1t, b1r, w2t, b2r, n_coords,
            s1, s2):
    nl = lines0.shape[0]
    n_blocks = n_coords // nl

    def body(occ_ref, l0_ref, l1_ref, l2_ref, w1t_ref, b1_ref, w2t_ref,
             b2_ref, out_ref):
        i = pl.program_id(0)
        hid0 = jnp.maximum(b1_ref[...], 0.0)
        const = jnp.dot(hid0, w2t_ref[...],
                        preferred_element_type=jnp.float32) + b2_ref[...]
        out_ref[...] = jnp.broadcast_to(const, out_ref.shape)

        @pl.when(i == 0)
        def _():
            occ = occ_ref[...]
            ox = jnp.max(occ[:, 0:NW], axis=1, keepdims=True)
            oy = jnp.max(occ[:, NW:2 * NW], axis=1, keepdims=True)
            oz = jnp.max(occ[:, 2 * NW:3 * NW], axis=1, keepdims=True)
            a = l0_ref[...] * ox
            b = l1_ref[...] * oy
            c = l2_ref[...] * oz
            ab = a * b
            h = a + b + c + (ab + (a + b) * c) * s1 + ab * c * s2
            hidden = jnp.maximum(
                jnp.dot(h, w1t_ref[...], preferred_element_type=jnp.float32)
                + b1_ref[...], 0.0)
            out_ref[...] = (jnp.dot(hidden, w2t_ref[...],
                                    preferred_element_type=jnp.float32)
                            + b2_ref[...])

    zero_map = lambda i: (0, 0)
    return pl.pallas_call(
        body,
        grid=(n_blocks,),
        in_specs=[
            pl.BlockSpec((nl, 3 * NW), zero_map),
            pl.BlockSpec((nl, lines0.shape[1]), zero_map),
            pl.BlockSpec((nl, lines0.shape[1]), zero_map),
            pl.BlockSpec((nl, lines0.shape[1]), zero_map),
            pl.BlockSpec(w1t.shape, zero_map),
            pl.BlockSpec(b1r.shape, zero_map),
            pl.BlockSpec(w2t.shape, zero_map),
            pl.BlockSpec(b2r.shape, zero_map),
        ],
        out_specs=pl.BlockSpec((nl, 1), lambda i: (i, 0)),
        out_shape=jax.ShapeDtypeStruct((n_coords, 1), jnp.float32),
    )(occ_cols, lines0, lines1, lines2, w1t, b1r, w2t, b2r)


def kernel(coordinates, lines0, lines1, lines2, W1, b1, W2, b2):
    batch, n_coords, _ = coordinates.shape
    nl, cl = lines0.shape
    total = batch * n_coords
    d = 4.0
    rng = 0.4

    coords_t = coordinates.reshape(total, 3).T.reshape(-1)
    pocc = _sc_occupancy(coords_t, nl, nl / d, d / 2.0)
    occ_cols = pocc.transpose(2, 0, 1).reshape(nl, 3 * NW)

    return _tc_mlp(occ_cols, lines0, lines1, lines2,
                   W1.T, b1.reshape(1, -1), W2.T, b2.reshape(1, -1),
                   n_coords, 1.0 / rng, 1.0 / rng ** 2)


# speedup vs baseline: 155.5360x; 1.3257x over previous
import functools

import jax
import jax.numpy as jnp
from jax import lax
from jax.experimental import pallas as pl
from jax.experimental.pallas import tpu as pltpu
from jax.experimental.pallas import tpu_sc as plsc

NC = 2
NS = 16
NW = NC * NS
L = 16

_RMAGIC = 12582912.0


def _sc_occupancy(coords_t, nl, scale, shift):
    total = coords_t.shape[0] // 3
    chunk = total // NW
    steps = chunk // L

    mesh = plsc.VectorSubcoreMesh(core_axis_name="c", subcore_axis_name="s")

    @functools.partial(
        pl.kernel,
        out_type=jax.ShapeDtypeStruct((3 * NW * nl,), jnp.float32),
        mesh=mesh,
        compiler_params=pltpu.CompilerParams(needs_layout_passes=False),
        scratch_types=[
            pltpu.VMEM((3 * chunk,), jnp.float32),
            pltpu.VMEM((3 * nl,), jnp.float32),
            pltpu.SemaphoreType.DMA,
        ],
    )
    def occ_kernel(coords_hbm, zeros_hbm, pocc_hbm, cbuf, occ, sem):
        wid = lax.axis_index("s") * NC + lax.axis_index("c")
        base = wid * chunk
        copies = [pltpu.async_copy(zeros_hbm, occ, sem)]
        for a in range(3):
            copies.append(
                pltpu.async_copy(coords_hbm.at[pl.ds(a * total + base, chunk)],
                                 cbuf.at[pl.ds(a * chunk, chunk)], sem))
        for cp in copies:
            cp.wait()

        ones16 = jnp.ones((L,), jnp.float32)
        hi = float(nl - 1)
        for a in range(3):

            @plsc.parallel_loop(0, steps, 1, unroll=8)
            def step(i, _a=a, _off=a * nl):
                x = cbuf[pl.ds(_a * chunk + i * L, L)]
                v = (x + shift) * scale
                vr = (v + _RMAGIC) - _RMAGIC
                mask = jnp.logical_and(vr >= 0.0, vr <= hi)
                idx = vr.astype(jnp.int32)
                if _off:
                    idx = idx + _off
                plsc.store_scatter(occ, [idx], ones16, mask=mask)

        for a in range(3):
            pltpu.sync_copy(occ.at[pl.ds(a * nl, nl)],
                            pocc_hbm.at[pl.ds((a * NW + wid) * nl, nl)])

    zeros = jnp.zeros((3 * nl,), jnp.float32)
    return occ_kernel(coords_t, zeros).reshape(3, NW, nl)


def _tc_mlp(occ_cols, lines0, lines1, lines2, w1t, b1r, w2t, b2r, n_coords,
            s1, s2):
    nl = lines0.shape[0]
    n_blocks = n_coords // nl

    def body(occ_ref, l0_ref, l1_ref, l2_ref, w1t_ref, b1_ref, w2t_ref,
             b2_ref, out_ref):
        i = pl.program_id(0)
        hid0 = jnp.maximum(b1_ref[...], 0.0)
        const = jnp.dot(hid0, w2t_ref[...],
                        preferred_element_type=jnp.float32) + b2_ref[...]
        out_ref[...] = jnp.broadcast_to(const, out_ref.shape)

        @pl.when(i == 0)
        def _():
            occ = occ_ref[...]
            ox = jnp.max(occ[:, 0:NW], axis=1, keepdims=True)
            oy = jnp.max(occ[:, NW:2 * NW], axis=1, keepdims=True)
            oz = jnp.max(occ[:, 2 * NW:3 * NW], axis=1, keepdims=True)
            a = l0_ref[...] * ox
            b = l1_ref[...] * oy
            c = l2_ref[...] * oz
            ab = a * b
            h = a + b + c + (ab + (a + b) * c) * s1 + ab * c * s2
            hidden = jnp.maximum(
                jnp.dot(h, w1t_ref[...], preferred_element_type=jnp.float32)
                + b1_ref[...], 0.0)
            out_ref[...] = (jnp.dot(hidden, w2t_ref[...],
                                    preferred_element_type=jnp.float32)
                            + b2_ref[...])

    zero_map = lambda i: (0, 0)
    return pl.pallas_call(
        body,
        grid=(n_blocks,),
        in_specs=[
            pl.BlockSpec((nl, 3 * NW), zero_map),
            pl.BlockSpec((nl, lines0.shape[1]), zero_map),
            pl.BlockSpec((nl, lines0.shape[1]), zero_map),
            pl.BlockSpec((nl, lines0.shape[1]), zero_map),
            pl.BlockSpec(w1t.shape, zero_map),
            pl.BlockSpec(b1r.shape, zero_map),
            pl.BlockSpec(w2t.shape, zero_map),
            pl.BlockSpec(b2r.shape, zero_map),
        ],
        out_specs=pl.BlockSpec((nl, 1), lambda i: (i, 0)),
        out_shape=jax.ShapeDtypeStruct((n_coords, 1), jnp.float32),
    )(occ_cols, lines0, lines1, lines2, w1t, b1r, w2t, b2r)


def kernel(coordinates, lines0, lines1, lines2, W1, b1, W2, b2):
    batch, n_coords, _ = coordinates.shape
    nl, cl = lines0.shape
    total = batch * n_coords
    d = 4.0
    rng = 0.4

    coords_t = coordinates.reshape(total, 3).T.reshape(-1)
    pocc = _sc_occupancy(coords_t, nl, nl / d, d / 2.0)
    occ_cols = pocc.transpose(2, 0, 1).reshape(nl, 3 * NW)

    return _tc_mlp(occ_cols, lines0, lines1, lines2,
                   W1.T, b1.reshape(1, -1), W2.T, b2.reshape(1, -1),
                   n_coords, 1.0 / rng, 1.0 / rng ** 2)


# speedup vs baseline: 243.2163x; 1.5637x over previous
import functools

import jax
import jax.numpy as jnp
from jax import lax
from jax.experimental import pallas as pl
from jax.experimental.pallas import tpu as pltpu
from jax.experimental.pallas import tpu_sc as plsc

NC = 2
NS = 16
NW = NC * NS
L = 16

_RMAGIC = 12582912.0


def _sc_occupancy(coords_t, nl, scale, shift):
    total = coords_t.shape[0] // 3
    chunk = total // NW
    steps = chunk // L

    mesh = plsc.VectorSubcoreMesh(core_axis_name="c", subcore_axis_name="s")

    @functools.partial(
        pl.kernel,
        out_type=jax.ShapeDtypeStruct((3 * NW * nl,), jnp.float32),
        mesh=mesh,
        compiler_params=pltpu.CompilerParams(needs_layout_passes=False),
        scratch_types=[
            pltpu.VMEM((3 * chunk,), jnp.float32),
            pltpu.VMEM((3 * nl,), jnp.float32),
            pltpu.SemaphoreType.DMA,
        ],
    )
    def occ_kernel(coords_hbm, zeros_hbm, pocc_hbm, cbuf, occ, sem):
        wid = lax.axis_index("s") * NC + lax.axis_index("c")
        base = wid * chunk
        copies = [pltpu.async_copy(zeros_hbm, occ, sem)]
        for a in range(3):
            copies.append(
                pltpu.async_copy(coords_hbm.at[pl.ds(a * total + base, chunk)],
                                 cbuf.at[pl.ds(a * chunk, chunk)], sem))
        for cp in copies:
            cp.wait()

        ones16 = jnp.ones((L,), jnp.float32)
        hi = float(nl - 1)
        for a in range(3):

            @plsc.parallel_loop(0, steps, 1, unroll=8)
            def step(i, _a=a, _off=a * nl):
                x = cbuf[pl.ds(_a * chunk + i * L, L)]
                v = (x + shift) * scale
                vr = (v + _RMAGIC) - _RMAGIC
                mask = jnp.logical_and(vr >= 0.0, vr <= hi)
                idx = vr.astype(jnp.int32)
                if _off:
                    idx = idx + _off
                plsc.store_scatter(occ, [idx], ones16, mask=mask)

        for a in range(3):
            pltpu.sync_copy(occ.at[pl.ds(a * nl, nl)],
                            pocc_hbm.at[pl.ds((a * NW + wid) * nl, nl)])

    zeros = jnp.zeros((3 * nl,), jnp.float32)
    return occ_kernel(coords_t, zeros).reshape(3 * NW, nl)


def _tc_mlp(occ_cols, lines0, lines1, lines2, w1t, b1r, w2t, b2r, n_coords,
            s1, s2):
    nl = lines0.shape[0]
    n_blocks = n_coords // nl

    def body(occ_ref, l0_ref, l1_ref, l2_ref, w1t_ref, b1_ref, w2t_ref,
             b2_ref, out_ref, const_ref):
        hid0 = jnp.maximum(b1_ref[...], 0.0)
        const_ref[...] = jnp.dot(hid0, w2t_ref[...],
                                 preferred_element_type=jnp.float32) + b2_ref[...]

        ones_w = jnp.ones((NW, 1), jnp.float32)
        dn = (((0,), (0,)), ((), ()))

        def colsum(rows):
            return lax.dot_general(rows, ones_w, dn,
                                   preferred_element_type=jnp.float32)

        occ = occ_ref[...]
        ox = jnp.minimum(colsum(occ[0:NW]), 1.0)
        oy = jnp.minimum(colsum(occ[NW:2 * NW]), 1.0)
        oz = jnp.minimum(colsum(occ[2 * NW:3 * NW]), 1.0)
        a = l0_ref[...] * ox
        b = l1_ref[...] * oy
        c = l2_ref[...] * oz
        ab = a * b
        h = a + b + c + (ab + (a + b) * c) * s1 + ab * c * s2
        hidden = jnp.maximum(
            jnp.dot(h, w1t_ref[...], preferred_element_type=jnp.float32)
            + b1_ref[...], 0.0)
        out_ref[...] = (jnp.dot(hidden, w2t_ref[...],
                                preferred_element_type=jnp.float32)
                        + b2_ref[...])

    out_main, const = pl.pallas_call(
        body,
        out_shape=[jax.ShapeDtypeStruct((nl, 1), jnp.float32),
                   jax.ShapeDtypeStruct((1, 1), jnp.float32)],
    )(occ_cols, lines0, lines1, lines2, w1t, b1r, w2t, b2r)
    tail = jnp.broadcast_to(const, (n_coords - nl, 1))
    return jnp.concatenate([out_main, tail], axis=0)


def kernel(coordinates, lines0, lines1, lines2, W1, b1, W2, b2):
    batch, n_coords, _ = coordinates.shape
    nl, cl = lines0.shape
    total = batch * n_coords
    d = 4.0
    rng = 0.4

    coords_t = coordinates.reshape(total, 3).T.reshape(-1)
    occ_rows = _sc_occupancy(coords_t, nl, nl / d, d / 2.0)

    return _tc_mlp(occ_rows, lines0, lines1, lines2,
                   W1.T, b1.reshape(1, -1), W2.T, b2.reshape(1, -1),
                   n_coords, 1.0 / rng, 1.0 / rng ** 2)


# speedup vs baseline: 244.0499x; 1.0034x over previous
import functools

import jax
import jax.numpy as jnp
from jax import lax
from jax.experimental import pallas as pl
from jax.experimental.pallas import tpu as pltpu
from jax.experimental.pallas import tpu_sc as plsc

NC = 2
NS = 16
NW = NC * NS
L = 16

_RMAGIC = 12582912.0


def _sc_occupancy(coords_t, nl, scale, shift):
    total = coords_t.shape[0] // 3
    chunk = total // NW
    steps = chunk // L

    mesh = plsc.VectorSubcoreMesh(core_axis_name="c", subcore_axis_name="s")

    blk = nl + 16
    log_steps = steps.bit_length() - 1
    assert steps == 1 << log_steps

    @functools.partial(
        pl.kernel,
        out_type=jax.ShapeDtypeStruct((3 * NW * nl,), jnp.float32),
        mesh=mesh,
        compiler_params=pltpu.CompilerParams(needs_layout_passes=False),
        scratch_types=[
            pltpu.VMEM((3 * chunk,), jnp.float32),
            pltpu.VMEM((3 * blk,), jnp.float32),
            pltpu.SemaphoreType.DMA,
        ],
    )
    def occ_kernel(coords_hbm, zeros_hbm, pocc_hbm, cbuf, occ, sem):
        wid = lax.axis_index("s") * NC + lax.axis_index("c")
        base = wid * chunk
        copies = [pltpu.async_copy(zeros_hbm, occ, sem)]
        for a in range(3):
            copies.append(
                pltpu.async_copy(coords_hbm.at[pl.ds(a * total + base, chunk)],
                                 cbuf.at[pl.ds(a * chunk, chunk)], sem))
        for cp in copies:
            cp.wait()

        ones16 = jnp.ones((L,), jnp.float32)

        @plsc.parallel_loop(0, 3 * steps, 1, unroll=16)
        def step(i):
            x = cbuf[pl.ds(i * L, L)]
            v = (x + shift) * scale
            vr = (v + _RMAGIC) - _RMAGIC
            vc = jnp.minimum(jnp.maximum(vr, -1.0), float(nl))
            axis_base = lax.shift_right_logical(i, log_steps) * blk + 8
            idx = vc.astype(jnp.int32) + axis_base
            plsc.store_scatter(occ, [idx], ones16)

        for a in range(3):
            pltpu.sync_copy(occ.at[pl.ds(a * blk + 8, nl)],
                            pocc_hbm.at[pl.ds((a * NW + wid) * nl, nl)])

    zeros = jnp.zeros((3 * blk,), jnp.float32)
    return occ_kernel(coords_t, zeros).reshape(3 * NW, nl)


def _tc_mlp(occ_cols, lines0, lines1, lines2, w1t, b1r, w2t, b2r, n_coords,
            s1, s2):
    nl = lines0.shape[0]
    n_blocks = n_coords // nl

    def body(occ_ref, l0_ref, l1_ref, l2_ref, w1t_ref, b1_ref, w2t_ref,
             b2_ref, out_ref, const_ref):
        hid0 = jnp.maximum(b1_ref[...], 0.0)
        const_ref[...] = jnp.dot(hid0, w2t_ref[...],
                                 preferred_element_type=jnp.float32) + b2_ref[...]

        ones_w = jnp.ones((NW, 1), jnp.float32)
        dn = (((0,), (0,)), ((), ()))

        def colsum(rows):
            return lax.dot_general(rows, ones_w, dn,
                                   preferred_element_type=jnp.float32)

        occ = occ_ref[...]
        ox = jnp.minimum(colsum(occ[0:NW]), 1.0)
        oy = jnp.minimum(colsum(occ[NW:2 * NW]), 1.0)
        oz = jnp.minimum(colsum(occ[2 * NW:3 * NW]), 1.0)
        a = l0_ref[...] * ox
        b = l1_ref[...] * oy
        c = l2_ref[...] * oz
        ab = a * b
        h = a + b + c + (ab + (a + b) * c) * s1 + ab * c * s2
        hidden = jnp.maximum(
            jnp.dot(h, w1t_ref[...], preferred_element_type=jnp.float32)
            + b1_ref[...], 0.0)
        out_ref[...] = (jnp.dot(hidden, w2t_ref[...],
                                preferred_element_type=jnp.float32)
                        + b2_ref[...])

    out_main, const = pl.pallas_call(
        body,
        out_shape=[jax.ShapeDtypeStruct((nl, 1), jnp.float32),
                   jax.ShapeDtypeStruct((1, 1), jnp.float32)],
    )(occ_cols, lines0, lines1, lines2, w1t, b1r, w2t, b2r)
    tail = jnp.broadcast_to(const, (n_coords - nl, 1))
    return jnp.concatenate([out_main, tail], axis=0)


def kernel(coordinates, lines0, lines1, lines2, W1, b1, W2, b2):
    batch, n_coords, _ = coordinates.shape
    nl, cl = lines0.shape
    total = batch * n_coords
    d = 4.0
    rng = 0.4

    coords_t = coordinates.reshape(total, 3).T.reshape(-1)
    occ_rows = _sc_occupancy(coords_t, nl, nl / d, d / 2.0)

    return _tc_mlp(occ_rows, lines0, lines1, lines2,
                   W1.T, b1.reshape(1, -1), W2.T, b2.reshape(1, -1),
                   n_coords, 1.0 / rng, 1.0 / rng ** 2)


# speedup vs baseline: 267.9390x; 1.0979x over previous
import functools

import jax
import jax.numpy as jnp
from jax import lax
from jax.experimental import pallas as pl
from jax.experimental.pallas import tpu as pltpu
from jax.experimental.pallas import tpu_sc as plsc

NC = 2
NS = 16
NW = NC * NS
L = 16

_RMAGIC = 12582912.0


def _sc_occupancy(coords_t, nl, scale, shift):
    total = coords_t.shape[0] // 3
    chunk = total // NW
    steps = chunk // L

    mesh = plsc.VectorSubcoreMesh(core_axis_name="c", subcore_axis_name="s")

    blk = nl + 16

    @functools.partial(
        pl.kernel,
        out_type=jax.ShapeDtypeStruct((3 * NW * nl,), jnp.float32),
        mesh=mesh,
        compiler_params=pltpu.CompilerParams(needs_layout_passes=False),
        scratch_types=[
            pltpu.VMEM((3 * chunk,), jnp.float32),
            pltpu.VMEM((3 * blk,), jnp.float32),
            pltpu.SemaphoreType.DMA,
        ],
    )
    def occ_kernel(coords_hbm, pocc_hbm, cbuf, occ, sem):
        wid = lax.axis_index("s") * NC + lax.axis_index("c")
        base = wid * chunk
        copies = []
        for a in range(3):
            copies.append(
                pltpu.async_copy(coords_hbm.at[pl.ds(a * total + base, chunk)],
                                 cbuf.at[pl.ds(a * chunk, chunk)], sem))

        zero16 = jnp.zeros((L,), jnp.float32)

        @plsc.parallel_loop(0, 3 * blk // L, 1, unroll=16)
        def zstep(i):
            occ[pl.ds(i * L, L)] = zero16

        ones16 = jnp.ones((L,), jnp.float32)
        for a in range(3):
            copies[a].wait()

            @plsc.parallel_loop(0, steps, 1, unroll=16)
            def step(i, _a=a, _base=a * blk + 8):
                x = cbuf[pl.ds(_a * chunk + i * L, L)]
                v = (x + shift) * scale
                vr = (v + _RMAGIC) - _RMAGIC
                vc = jnp.minimum(jnp.maximum(vr, -1.0), float(nl))
                idx = vc.astype(jnp.int32) + _base
                plsc.store_scatter(occ, [idx], ones16)

        for a in range(3):
            pltpu.sync_copy(occ.at[pl.ds(a * blk + 8, nl)],
                            pocc_hbm.at[pl.ds((a * NW + wid) * nl, nl)])

    return occ_kernel(coords_t).reshape(3 * NW, nl)


def _tc_mlp(occ_cols, lines0, lines1, lines2, w1t, b1r, w2t, b2r, n_coords,
            s1, s2):
    nl = lines0.shape[0]
    n_blocks = n_coords // nl

    def body(occ_ref, l0_ref, l1_ref, l2_ref, w1t_ref, b1_ref, w2t_ref,
             b2_ref, out_ref, const_ref):
        hid0 = jnp.maximum(b1_ref[...], 0.0)
        const_ref[...] = jnp.dot(hid0, w2t_ref[...],
                                 preferred_element_type=jnp.float32) + b2_ref[...]

        ones_w = jnp.ones((NW, 1), jnp.float32)
        dn = (((0,), (0,)), ((), ()))

        def colsum(rows):
            return lax.dot_general(rows, ones_w, dn,
                                   preferred_element_type=jnp.float32)

        occ = occ_ref[...]
        ox = jnp.minimum(colsum(occ[0:NW]), 1.0)
        oy = jnp.minimum(colsum(occ[NW:2 * NW]), 1.0)
        oz = jnp.minimum(colsum(occ[2 * NW:3 * NW]), 1.0)
        a = l0_ref[...] * ox
        b = l1_ref[...] * oy
        c = l2_ref[...] * oz
        ab = a * b
        h = a + b + c + (ab + (a + b) * c) * s1 + ab * c * s2
        hidden = jnp.maximum(
            jnp.dot(h, w1t_ref[...], preferred_element_type=jnp.float32)
            + b1_ref[...], 0.0)
        out_ref[...] = (jnp.dot(hidden, w2t_ref[...],
                                preferred_element_type=jnp.float32)
                        + b2_ref[...])

    out_main, const = pl.pallas_call(
        body,
        out_shape=[jax.ShapeDtypeStruct((nl, 1), jnp.float32),
                   jax.ShapeDtypeStruct((1, 1), jnp.float32)],
    )(occ_cols, lines0, lines1, lines2, w1t, b1r, w2t, b2r)
    tail = jnp.broadcast_to(const, (n_coords - nl, 1))
    return jnp.concatenate([out_main, tail], axis=0)


def kernel(coordinates, lines0, lines1, lines2, W1, b1, W2, b2):
    batch, n_coords, _ = coordinates.shape
    nl, cl = lines0.shape
    total = batch * n_coords
    d = 4.0
    rng = 0.4

    coords_t = coordinates.reshape(total, 3).T.reshape(-1)
    occ_rows = _sc_occupancy(coords_t, nl, nl / d, d / 2.0)

    return _tc_mlp(occ_rows, lines0, lines1, lines2,
                   W1.T, b1.reshape(1, -1), W2.T, b2.reshape(1, -1),
                   n_coords, 1.0 / rng, 1.0 / rng ** 2)
